# baseline (speedup 1.0000x reference)
import functools

import numpy as np

import jax
import jax.numpy as jnp
from jax import lax
from jax.experimental import pallas as pl
from jax.experimental.pallas import tpu as pltpu
from jax.experimental.pallas import tpu_sc as plsc

N = 10000
NUM_GRAPHS = 64
NC, NS, LANES = 2, 16, 16
NW = NC * NS
CHUNK = 128
NBUF = 8
SDIST = 4
N_PAD = 10112
STRIPE = N_PAD // NS

E = 320000
ROWS = E // CHUNK
CH = ROWS // NW
XTRA = ROWS - CH * NW

_Z16 = np.zeros((N_PAD, 16), np.float32)
_Z32 = np.zeros((N_PAD, 32), np.float32)
_Z64 = np.zeros((N_PAD, 64), np.float32)


def _mesh():
    return plsc.VectorSubcoreMesh(
        core_axis_name="c", subcore_axis_name="s",
        num_cores=NC, num_subcores=NS)


_SC_PARAMS = pltpu.CompilerParams(use_tc_tiling_on_sc=False)


@functools.lru_cache(maxsize=None)
def _deg_kernel():
    @functools.partial(
        pl.kernel,
        out_type=jax.ShapeDtypeStruct((NC, N_PAD, LANES), jnp.float32),
        mesh=_mesh(),
        compiler_params=_SC_PARAMS,
        scratch_types=[
            pltpu.VMEM((CH * CHUNK,), jnp.int32),
            pltpu.VMEM((CHUNK,), jnp.int32),
            pltpu.VMEM((CHUNK, LANES), jnp.float32),
            pltpu.VMEM_SHARED((N_PAD, LANES), jnp.float32),
            pltpu.SemaphoreType.DMA,
        ],
    )
    def deg(ei_hbm, zero_hbm, out_hbm, didx, didxx, ones_v, acc, sem):
        cid = lax.axis_index("c")
        sid = lax.axis_index("s")
        wid = cid * NS + sid
        pltpu.sync_copy(ei_hbm.at[1, pl.ds(wid * CH * CHUNK, CH * CHUNK)], didx)
        for r in range(CHUNK):
            ones_v[r, :] = jnp.ones((LANES,), jnp.float32)
        r0 = sid * STRIPE
        pltpu.sync_copy(zero_hbm.at[pl.ds(r0, STRIPE)], acc.at[pl.ds(r0, STRIPE)])
        plsc.subcore_barrier()
        handles = [pltpu.async_copy(ones_v, acc.at[didx.at[pl.ds(j * CHUNK, CHUNK)]],
                                    sem, add=True)
                   for j in range(CH)]
        for h in handles:
            h.wait()

        @pl.when(wid < XTRA)
        def _extra():
            pltpu.sync_copy(
                ei_hbm.at[1, pl.ds((NW * CH + wid) * CHUNK, CHUNK)], didxx)
            pltpu.sync_copy(ones_v, acc.at[didxx], add=True)

        plsc.subcore_barrier()
        pltpu.sync_copy(acc.at[pl.ds(r0, STRIPE)],
                        out_hbm.at[cid, pl.ds(r0, STRIPE)])
    return deg


@functools.lru_cache(maxsize=None)
def _agg_kernel(d):
    @functools.partial(
        pl.kernel,
        out_type=jax.ShapeDtypeStruct((NC, N_PAD, d), jnp.float32),
        mesh=_mesh(),
        compiler_params=_SC_PARAMS,
        scratch_types=[
            pltpu.VMEM((CH * CHUNK,), jnp.int32),
            pltpu.VMEM((CH * CHUNK,), jnp.int32),
            pltpu.VMEM((CHUNK,), jnp.int32),
            pltpu.VMEM((CHUNK,), jnp.int32),
            pltpu.VMEM((NBUF, CHUNK, d), jnp.float32),
            pltpu.VMEM_SHARED((N_PAD, d), jnp.float32),
            [pltpu.SemaphoreType.DMA] * NBUF,
            [pltpu.SemaphoreType.DMA] * NBUF,
        ],
    )
    def agg(ei_hbm, p_hbm, zero_hbm, out_hbm,
            sidx, didx, sidxx, didxx, rows, acc, gsems, ssems):
        cid = lax.axis_index("c")
        sid = lax.axis_index("s")
        wid = cid * NS + sid
        e0 = wid * CH * CHUNK
        pltpu.sync_copy(ei_hbm.at[0, pl.ds(e0, CH * CHUNK)], sidx)
        pltpu.sync_copy(ei_hbm.at[1, pl.ds(e0, CH * CHUNK)], didx)
        r0 = sid * STRIPE
        pltpu.sync_copy(zero_hbm.at[pl.ds(r0, STRIPE)], acc.at[pl.ds(r0, STRIPE)])
        plsc.subcore_barrier()

        def gather(k):
            return pltpu.async_copy(
                p_hbm.at[sidx.at[pl.ds(k * CHUNK, CHUNK)]],
                rows.at[k % NBUF], gsems[k % NBUF])

        gh = [None] * NBUF
        sh = [None] * NBUF
        for k in range(min(NBUF - SDIST, CH)):
            gh[k % NBUF] = gather(k)
        for j in range(CH):
            b = j % NBUF
            k = j + NBUF - SDIST
            if k < CH:
                bk = k % NBUF
                if sh[bk] is not None:
                    sh[bk].wait()
                gh[bk] = gather(k)
            gh[b].wait()
            sh[b] = pltpu.async_copy(
                rows.at[b], acc.at[didx.at[pl.ds(j * CHUNK, CHUNK)]],
                ssems[b], add=True)
        for j in range(max(0, CH - NBUF), CH):
            sh[j % NBUF].wait()

        @pl.when(wid < XTRA)
        def _extra():
            x0 = (NW * CH + wid) * CHUNK
            pltpu.sync_copy(ei_hbm.at[0, pl.ds(x0, CHUNK)], sidxx)
            pltpu.sync_copy(ei_hbm.at[1, pl.ds(x0, CHUNK)], didxx)
            pltpu.async_copy(p_hbm.at[sidxx], rows.at[0], gsems[0]).wait()
            pltpu.sync_copy(rows.at[0], acc.at[didxx], add=True)

        plsc.subcore_barrier()
        pltpu.sync_copy(acc.at[pl.ds(r0, STRIPE)],
                        out_hbm.at[cid, pl.ds(r0, STRIPE)])
    return agg


GB = 8
RB = N_PAD // GB


def _rowmask(i):
    row = i * RB + lax.broadcasted_iota(jnp.int32, (RB, 1), 0)
    return row < N


def _tc_pre(degp_ref, x_ref, w_ref, p_ref, dinv_ref):
    i = pl.program_id(0)
    degp = degp_ref[...]
    deg = degp[0, :, 0:1] + degp[1, :, 0:1] + 1.0
    mask = _rowmask(i)
    dinv = jnp.where(mask, lax.rsqrt(deg), 0.0)
    q = jnp.dot(x_ref[...], w_ref[...], preferred_element_type=jnp.float32)
    p_ref[...] = jnp.where(mask, q * dinv, 0.0)
    dinv_ref[...] = jnp.broadcast_to(dinv, (RB, 16))


def _tc_mid(dinv_ref, agg_ref, p_ref, b_ref, w_ref, o_ref):
    dinv = dinv_ref[...][:, 0:1]
    a = agg_ref[...]
    z = jnp.maximum((a[0] + a[1] + p_ref[...]) * dinv + b_ref[...], 0.0)
    o_ref[...] = jnp.dot(z, w_ref[...],
                         preferred_element_type=jnp.float32) * dinv


def _tc_fin(dinv_ref, agg_ref, p_ref, b_ref, bat_ref, wfc_ref, bfc_ref,
            o_ref, sums_acc, cnt_acc):
    i = pl.program_id(0)
    dinv = dinv_ref[...][:, 0:1]
    a = agg_ref[...]
    z = jnp.maximum((a[0] + a[1] + p_ref[...]) * dinv + b_ref[...], 0.0)
    mask = _rowmask(i)
    z = jnp.where(mask, z, 0.0)
    g = lax.broadcasted_iota(jnp.int32, (NUM_GRAPHS, RB), 0)
    oh = (g == bat_ref[...][0, 0][None, :]).astype(jnp.float32)
    oh = jnp.where(mask[None, :, 0], oh, 0.0)
    sums = jnp.dot(oh, z, preferred_element_type=jnp.float32)
    cnt = jnp.sum(oh, axis=1, keepdims=True)

    @pl.when(i == 0)
    def _init():
        sums_acc[...] = sums
        cnt_acc[...] = cnt

    @pl.when(i > 0)
    def _accum():
        sums_acc[...] += sums
        cnt_acc[...] += cnt

    @pl.when(i == GB - 1)
    def _finish():
        pooled = sums_acc[...] / jnp.maximum(cnt_acc[...], 1.0)
        o_ref[...] = jax.nn.sigmoid(
            jnp.dot(pooled, wfc_ref[...], preferred_element_type=jnp.float32)
            + bfc_ref[...])


def _row_blk(*trail):
    return pl.BlockSpec((RB,) + trail, lambda i: (i,) + (0,) * len(trail))


def _pair_blk(d):
    return pl.BlockSpec((2, RB, d), lambda i: (0, i, 0))


def _full(shape):
    return pl.BlockSpec(shape, lambda i: (0,) * len(shape))


def kernel(x, edge_index, batch, W1, b1, W2, b2, W3, b3, Wfc, bfc):
    ei = edge_index.astype(jnp.int32)
    bat = jnp.concatenate(
        [batch.astype(jnp.int32),
         jnp.full((N_PAD - N,), NUM_GRAPHS, jnp.int32)]).reshape(GB, 1, RB)

    degp = _deg_kernel()(ei, _Z16)

    p1, dinv16 = pl.pallas_call(
        _tc_pre,
        grid=(GB,),
        in_specs=[_pair_blk(16), _row_blk(128), _full((128, 64))],
        out_specs=[_row_blk(64), _row_blk(16)],
        out_shape=[jax.ShapeDtypeStruct((N_PAD, 64), jnp.float32),
                   jax.ShapeDtypeStruct((N_PAD, 16), jnp.float32)],
    )(degp, x, W1)
    a1 = _agg_kernel(64)(ei, p1, _Z64)
    p2 = pl.pallas_call(
        _tc_mid,
        grid=(GB,),
        in_specs=[_row_blk(16), _pair_blk(64), _row_blk(64),
                  _full((64,)), _full((64, 32))],
        out_specs=_row_blk(32),
        out_shape=jax.ShapeDtypeStruct((N_PAD, 32), jnp.float32),
    )(dinv16, a1, p1, b1, W2)
    a2 = _agg_kernel(32)(ei, p2, _Z32)
    p3 = pl.pallas_call(
        _tc_mid,
        grid=(GB,),
        in_specs=[_row_blk(16), _pair_blk(32), _row_blk(32),
                  _full((32,)), _full((32, 16))],
        out_specs=_row_blk(16),
        out_shape=jax.ShapeDtypeStruct((N_PAD, 16), jnp.float32),
    )(dinv16, a2, p2, b2, W3)
    a3 = _agg_kernel(16)(ei, p3, _Z16)
    out = pl.pallas_call(
        _tc_fin,
        grid=(GB,),
        in_specs=[_row_blk(16), _pair_blk(16), _row_blk(16),
                  _full((16,)), pl.BlockSpec((1, 1, RB), lambda i: (i, 0, 0)),
                  _full((16, 1)), _full((1,))],
        out_specs=_full((NUM_GRAPHS, 1)),
        out_shape=jax.ShapeDtypeStruct((NUM_GRAPHS, 1), jnp.float32),
        scratch_shapes=[pltpu.VMEM((NUM_GRAPHS, 16), jnp.float32),
                        pltpu.VMEM((NUM_GRAPHS, 1), jnp.float32)],
    )(dinv16, a3, p3, b3, bat, Wfc, bfc)
    return out


# speedup vs baseline: 1.7093x; 1.7093x over previous
import functools

import numpy as np

import jax
import jax.numpy as jnp
from jax import lax
from jax.experimental import pallas as pl
from jax.experimental.pallas import tpu as pltpu
from jax.experimental.pallas import tpu_sc as plsc

N = 10000
NUM_GRAPHS = 64
NC, NS, LANES = 2, 16, 16
NW = NC * NS
CHUNK = 128
NBUF = 8
SDIST = 4
N_PAD = 10112
STRIPE = N_PAD // NS

E = 320000
ROWS = E // CHUNK
CH = ROWS // NW
XTRA = ROWS - CH * NW

_Z16 = np.zeros((N_PAD, 16), np.float32)
_Z32 = np.zeros((N_PAD, 32), np.float32)
_Z64 = np.zeros((N_PAD, 64), np.float32)


def _mesh():
    return plsc.VectorSubcoreMesh(
        core_axis_name="c", subcore_axis_name="s",
        num_cores=NC, num_subcores=NS)


_SC_PARAMS = pltpu.CompilerParams(use_tc_tiling_on_sc=False)


@functools.lru_cache(maxsize=None)
def _deg_kernel():
    @functools.partial(
        pl.kernel,
        out_type=jax.ShapeDtypeStruct((NC, N_PAD, LANES), jnp.float32),
        mesh=_mesh(),
        compiler_params=_SC_PARAMS,
        scratch_types=[
            pltpu.VMEM((CH * CHUNK,), jnp.int32),
            pltpu.VMEM((CHUNK,), jnp.int32),
            pltpu.VMEM((CHUNK, LANES), jnp.float32),
            pltpu.VMEM_SHARED((N_PAD, LANES), jnp.float32),
            pltpu.SemaphoreType.DMA,
        ],
    )
    def deg(ei_hbm, zero_hbm, out_hbm, didx, didxx, ones_v, acc, sem):
        cid = lax.axis_index("c")
        sid = lax.axis_index("s")
        wid = cid * NS + sid
        pltpu.sync_copy(ei_hbm.at[1, pl.ds(wid * CH * CHUNK, CH * CHUNK)], didx)
        for r in range(CHUNK):
            ones_v[r, :] = jnp.ones((LANES,), jnp.float32)
        r0 = sid * STRIPE
        pltpu.sync_copy(zero_hbm.at[pl.ds(r0, STRIPE)], acc.at[pl.ds(r0, STRIPE)])
        plsc.subcore_barrier()
        handles = [pltpu.async_copy(ones_v, acc.at[didx.at[pl.ds(j * CHUNK, CHUNK)]],
                                    sem, add=True)
                   for j in range(CH)]
        for h in handles:
            h.wait()

        @pl.when(wid < XTRA)
        def _extra():
            pltpu.sync_copy(
                ei_hbm.at[1, pl.ds((NW * CH + wid) * CHUNK, CHUNK)], didxx)
            pltpu.sync_copy(ones_v, acc.at[didxx], add=True)

        plsc.subcore_barrier()
        pltpu.sync_copy(acc.at[pl.ds(r0, STRIPE)],
                        out_hbm.at[cid, pl.ds(r0, STRIPE)])
    return deg


@functools.lru_cache(maxsize=None)
def _agg_kernel(d):
    @functools.partial(
        pl.kernel,
        out_type=jax.ShapeDtypeStruct((NC, N_PAD, d), jnp.float32),
        mesh=_mesh(),
        compiler_params=_SC_PARAMS,
        scratch_types=[
            pltpu.VMEM((CH * CHUNK,), jnp.int32),
            pltpu.VMEM((CH * CHUNK,), jnp.int32),
            pltpu.VMEM((CHUNK,), jnp.int32),
            pltpu.VMEM((CHUNK,), jnp.int32),
            pltpu.VMEM((NBUF, CHUNK, d), jnp.float32),
            pltpu.VMEM_SHARED((N_PAD, d), jnp.float32),
            [pltpu.SemaphoreType.DMA] * NBUF,
            [pltpu.SemaphoreType.DMA] * NBUF,
        ],
    )
    def agg(ei_hbm, p_hbm, zero_hbm, out_hbm,
            sidx, didx, sidxx, didxx, rows, acc, gsems, ssems):
        cid = lax.axis_index("c")
        sid = lax.axis_index("s")
        wid = cid * NS + sid
        e0 = wid * CH * CHUNK
        pltpu.sync_copy(ei_hbm.at[0, pl.ds(e0, CH * CHUNK)], sidx)
        pltpu.sync_copy(ei_hbm.at[1, pl.ds(e0, CH * CHUNK)], didx)
        r0 = sid * STRIPE
        pltpu.sync_copy(zero_hbm.at[pl.ds(r0, STRIPE)], acc.at[pl.ds(r0, STRIPE)])
        plsc.subcore_barrier()

        def gather(k):
            return pltpu.async_copy(
                p_hbm.at[sidx.at[pl.ds(k * CHUNK, CHUNK)]],
                rows.at[k % NBUF], gsems[k % NBUF])

        gh = [None] * NBUF
        sh = [None] * NBUF
        for k in range(min(NBUF - SDIST, CH)):
            gh[k % NBUF] = gather(k)
        for j in range(CH):
            b = j % NBUF
            k = j + NBUF - SDIST
            if k < CH:
                bk = k % NBUF
                if sh[bk] is not None:
                    sh[bk].wait()
                gh[bk] = gather(k)
            gh[b].wait()
            sh[b] = pltpu.async_copy(
                rows.at[b], acc.at[didx.at[pl.ds(j * CHUNK, CHUNK)]],
                ssems[b], add=True)
        for j in range(max(0, CH - NBUF), CH):
            sh[j % NBUF].wait()

        @pl.when(wid < XTRA)
        def _extra():
            x0 = (NW * CH + wid) * CHUNK
            pltpu.sync_copy(ei_hbm.at[0, pl.ds(x0, CHUNK)], sidxx)
            pltpu.sync_copy(ei_hbm.at[1, pl.ds(x0, CHUNK)], didxx)
            pltpu.async_copy(p_hbm.at[sidxx], rows.at[0], gsems[0]).wait()
            pltpu.sync_copy(rows.at[0], acc.at[didxx], add=True)

        plsc.subcore_barrier()
        pltpu.sync_copy(acc.at[pl.ds(r0, STRIPE)],
                        out_hbm.at[cid, pl.ds(r0, STRIPE)])
    return agg


GB = 8
RB = N_PAD // GB


def _rowmask(i):
    row = i * RB + lax.broadcasted_iota(jnp.int32, (RB, 1), 0)
    return row < N


def _tc_pre(degp_ref, x_ref, w_ref, p_ref, dinv_ref):
    i = pl.program_id(0)
    degp = degp_ref[...]
    deg = degp[0, :, 0:1] + degp[1, :, 0:1] + 1.0
    mask = _rowmask(i)
    dinv = jnp.where(mask, lax.rsqrt(deg), 0.0)
    q = jnp.dot(x_ref[...], w_ref[...], preferred_element_type=jnp.float32)
    p_ref[...] = jnp.where(mask, q * dinv, 0.0)
    dinv_ref[...] = jnp.broadcast_to(dinv, (RB, 16))


def _tc_mid(dinv_ref, agg_ref, p_ref, b_ref, w_ref, o_ref):
    dinv = dinv_ref[...][:, 0:1]
    a = agg_ref[...]
    z = jnp.maximum((a[0] + a[1] + p_ref[...]) * dinv + b_ref[...], 0.0)
    o_ref[...] = jnp.dot(z, w_ref[...],
                         preferred_element_type=jnp.float32) * dinv


def _tc_fin(dinv_ref, agg_ref, p_ref, b_ref, bat_ref, wfc_ref, bfc_ref, o_ref):
    dinv = dinv_ref[...][:, 0:1]
    a = agg_ref[...]
    z = jnp.maximum((a[0] + a[1] + p_ref[...]) * dinv + b_ref[...], 0.0)
    row = lax.broadcasted_iota(jnp.int32, (N_PAD, 1), 0)
    z = jnp.where(row < N, z, 0.0)
    bat = jnp.concatenate(
        [bat_ref[...], jnp.full((N_PAD - N,), NUM_GRAPHS, jnp.int32)])
    g = lax.broadcasted_iota(jnp.int32, (NUM_GRAPHS, N_PAD), 0)
    oh = (g == bat[None, :]).astype(jnp.float32)
    sums = jnp.dot(oh, z, preferred_element_type=jnp.float32)
    cnt = jnp.sum(oh, axis=1, keepdims=True)
    pooled = sums / jnp.maximum(cnt, 1.0)
    o_ref[...] = jax.nn.sigmoid(
        jnp.dot(pooled, wfc_ref[...], preferred_element_type=jnp.float32)
        + bfc_ref[...])


def _row_blk(*trail):
    return pl.BlockSpec((RB,) + trail, lambda i: (i,) + (0,) * len(trail))


def _pair_blk(d):
    return pl.BlockSpec((2, RB, d), lambda i: (0, i, 0))


def _full(shape):
    return pl.BlockSpec(shape, lambda i: (0,) * len(shape))


def kernel(x, edge_index, batch, W1, b1, W2, b2, W3, b3, Wfc, bfc):
    ei = edge_index.astype(jnp.int32)
    bat = batch.astype(jnp.int32)

    degp = _deg_kernel()(ei, _Z16)

    p1, dinv16 = pl.pallas_call(
        _tc_pre,
        grid=(GB,),
        in_specs=[_pair_blk(16), _row_blk(128), _full((128, 64))],
        out_specs=[_row_blk(64), _row_blk(16)],
        out_shape=[jax.ShapeDtypeStruct((N_PAD, 64), jnp.float32),
                   jax.ShapeDtypeStruct((N_PAD, 16), jnp.float32)],
    )(degp, x, W1)
    a1 = _agg_kernel(64)(ei, p1, _Z64)
    p2 = pl.pallas_call(
        _tc_mid,
        grid=(GB,),
        in_specs=[_row_blk(16), _pair_blk(64), _row_blk(64),
                  _full((64,)), _full((64, 32))],
        out_specs=_row_blk(32),
        out_shape=jax.ShapeDtypeStruct((N_PAD, 32), jnp.float32),
    )(dinv16, a1, p1, b1, W2)
    a2 = _agg_kernel(32)(ei, p2, _Z32)
    p3 = pl.pallas_call(
        _tc_mid,
        grid=(GB,),
        in_specs=[_row_blk(16), _pair_blk(32), _row_blk(32),
                  _full((32,)), _full((32, 16))],
        out_specs=_row_blk(16),
        out_shape=jax.ShapeDtypeStruct((N_PAD, 16), jnp.float32),
    )(dinv16, a2, p2, b2, W3)
    a3 = _agg_kernel(16)(ei, p3, _Z16)
    out = pl.pallas_call(
        _tc_fin, out_shape=jax.ShapeDtypeStruct((NUM_GRAPHS, 1), jnp.float32),
    )(dinv16, a3, p3, b3, bat, Wfc, bfc)
    return out


# speedup vs baseline: 1.9760x; 1.1561x over previous
import functools

import ml_dtypes
import numpy as np

import jax
import jax.numpy as jnp
from jax import lax
from jax.experimental import pallas as pl
from jax.experimental.pallas import tpu as pltpu
from jax.experimental.pallas import tpu_sc as plsc

N = 10000
NUM_GRAPHS = 64
NC, NS, LANES = 2, 16, 16
NW = NC * NS
CHUNK = 128
NBUF = 8
SDIST = 4
N_PAD = 10112
STRIPE = N_PAD // NS

E = 320000
ROWS = E // CHUNK
CH = ROWS // NW
XTRA = ROWS - CH * NW

_Z16 = np.zeros((N_PAD, 16), np.float32)
_Z16B = np.zeros((N_PAD, 16), ml_dtypes.bfloat16)
_Z32B = np.zeros((N_PAD, 32), ml_dtypes.bfloat16)
_Z64B = np.zeros((N_PAD, 64), ml_dtypes.bfloat16)


def _mesh():
    return plsc.VectorSubcoreMesh(
        core_axis_name="c", subcore_axis_name="s",
        num_cores=NC, num_subcores=NS)


_SC_PARAMS = pltpu.CompilerParams(use_tc_tiling_on_sc=False)


@functools.lru_cache(maxsize=None)
def _deg_kernel():
    @functools.partial(
        pl.kernel,
        out_type=jax.ShapeDtypeStruct((NC, N_PAD, LANES), jnp.float32),
        mesh=_mesh(),
        compiler_params=_SC_PARAMS,
        scratch_types=[
            pltpu.VMEM((CH * CHUNK,), jnp.int32),
            pltpu.VMEM((CHUNK,), jnp.int32),
            pltpu.VMEM((CHUNK, LANES), jnp.float32),
            pltpu.VMEM_SHARED((N_PAD, LANES), jnp.float32),
            pltpu.SemaphoreType.DMA,
        ],
    )
    def deg(ei_hbm, zero_hbm, out_hbm, didx, didxx, ones_v, acc, sem):
        cid = lax.axis_index("c")
        sid = lax.axis_index("s")
        wid = cid * NS + sid
        pltpu.sync_copy(ei_hbm.at[1, pl.ds(wid * CH * CHUNK, CH * CHUNK)], didx)
        for r in range(CHUNK):
            ones_v[r, :] = jnp.ones((LANES,), jnp.float32)
        r0 = sid * STRIPE
        pltpu.sync_copy(zero_hbm.at[pl.ds(r0, STRIPE)], acc.at[pl.ds(r0, STRIPE)])
        plsc.subcore_barrier()
        handles = [pltpu.async_copy(ones_v, acc.at[didx.at[pl.ds(j * CHUNK, CHUNK)]],
                                    sem, add=True)
                   for j in range(CH)]
        for h in handles:
            h.wait()

        @pl.when(wid < XTRA)
        def _extra():
            pltpu.sync_copy(
                ei_hbm.at[1, pl.ds((NW * CH + wid) * CHUNK, CHUNK)], didxx)
            pltpu.sync_copy(ones_v, acc.at[didxx], add=True)

        plsc.subcore_barrier()
        pltpu.sync_copy(acc.at[pl.ds(r0, STRIPE)],
                        out_hbm.at[cid, pl.ds(r0, STRIPE)])
    return deg


@functools.lru_cache(maxsize=None)
def _agg_kernel(d):
    @functools.partial(
        pl.kernel,
        out_type=jax.ShapeDtypeStruct((NC, N_PAD, d), jnp.bfloat16),
        mesh=_mesh(),
        compiler_params=_SC_PARAMS,
        scratch_types=[
            pltpu.VMEM((CH * CHUNK,), jnp.int32),
            pltpu.VMEM((CH * CHUNK,), jnp.int32),
            pltpu.VMEM((CHUNK,), jnp.int32),
            pltpu.VMEM((CHUNK,), jnp.int32),
            pltpu.VMEM((NBUF, CHUNK, d), jnp.bfloat16),
            pltpu.VMEM_SHARED((N_PAD, d), jnp.bfloat16),
            [pltpu.SemaphoreType.DMA] * NBUF,
            [pltpu.SemaphoreType.DMA] * NBUF,
        ],
    )
    def agg(ei_hbm, p_hbm, zero_hbm, out_hbm,
            sidx, didx, sidxx, didxx, rows, acc, gsems, ssems):
        cid = lax.axis_index("c")
        sid = lax.axis_index("s")
        wid = cid * NS + sid
        e0 = wid * CH * CHUNK
        pltpu.sync_copy(ei_hbm.at[0, pl.ds(e0, CH * CHUNK)], sidx)
        pltpu.sync_copy(ei_hbm.at[1, pl.ds(e0, CH * CHUNK)], didx)
        r0 = sid * STRIPE
        pltpu.sync_copy(zero_hbm.at[pl.ds(r0, STRIPE)], acc.at[pl.ds(r0, STRIPE)])
        plsc.subcore_barrier()

        def gather(k):
            return pltpu.async_copy(
                p_hbm.at[sidx.at[pl.ds(k * CHUNK, CHUNK)]],
                rows.at[k % NBUF], gsems[k % NBUF])

        gh = [None] * NBUF
        sh = [None] * NBUF
        for k in range(min(NBUF - SDIST, CH)):
            gh[k % NBUF] = gather(k)
        for j in range(CH):
            b = j % NBUF
            k = j + NBUF - SDIST
            if k < CH:
                bk = k % NBUF
                if sh[bk] is not None:
                    sh[bk].wait()
                gh[bk] = gather(k)
            gh[b].wait()
            sh[b] = pltpu.async_copy(
                rows.at[b], acc.at[didx.at[pl.ds(j * CHUNK, CHUNK)]],
                ssems[b], add=True)
        for j in range(max(0, CH - NBUF), CH):
            sh[j % NBUF].wait()

        @pl.when(wid < XTRA)
        def _extra():
            x0 = (NW * CH + wid) * CHUNK
            pltpu.sync_copy(ei_hbm.at[0, pl.ds(x0, CHUNK)], sidxx)
            pltpu.sync_copy(ei_hbm.at[1, pl.ds(x0, CHUNK)], didxx)
            pltpu.async_copy(p_hbm.at[sidxx], rows.at[0], gsems[0]).wait()
            pltpu.sync_copy(rows.at[0], acc.at[didxx], add=True)

        plsc.subcore_barrier()
        pltpu.sync_copy(acc.at[pl.ds(r0, STRIPE)],
                        out_hbm.at[cid, pl.ds(r0, STRIPE)])
    return agg


GB = 8
RB = N_PAD // GB


def _rowmask(i):
    row = i * RB + lax.broadcasted_iota(jnp.int32, (RB, 1), 0)
    return row < N


def _tc_pre(degp_ref, x_ref, w_ref, p_ref, dinv_ref):
    i = pl.program_id(0)
    degp = degp_ref[...]
    deg = degp[0, :, 0:1] + degp[1, :, 0:1] + 1.0
    mask = _rowmask(i)
    dinv = jnp.where(mask, lax.rsqrt(deg), 0.0)
    q = jnp.dot(x_ref[...], w_ref[...], preferred_element_type=jnp.float32)
    p_ref[...] = jnp.where(mask, q * dinv, 0.0).astype(jnp.bfloat16)
    dinv_ref[...] = jnp.broadcast_to(dinv, (RB, 16))


def _tc_mid(dinv_ref, agg_ref, p_ref, b_ref, w_ref, o_ref):
    dinv = dinv_ref[...][:, 0:1]
    a = agg_ref[...].astype(jnp.float32)
    p = p_ref[...].astype(jnp.float32)
    z = jnp.maximum((a[0] + a[1] + p) * dinv + b_ref[...], 0.0)
    o_ref[...] = (jnp.dot(z, w_ref[...], preferred_element_type=jnp.float32)
                  * dinv).astype(jnp.bfloat16)


def _tc_fin(dinv_ref, agg_ref, p_ref, b_ref, bat_ref, wfc_ref, bfc_ref, o_ref):
    dinv = dinv_ref[...][:, 0:1]
    a = agg_ref[...].astype(jnp.float32)
    p = p_ref[...].astype(jnp.float32)
    z = jnp.maximum((a[0] + a[1] + p) * dinv + b_ref[...], 0.0)
    row = lax.broadcasted_iota(jnp.int32, (N_PAD, 1), 0)
    z = jnp.where(row < N, z, 0.0)
    bat = jnp.concatenate(
        [bat_ref[...], jnp.full((N_PAD - N,), NUM_GRAPHS, jnp.int32)])
    g = lax.broadcasted_iota(jnp.int32, (NUM_GRAPHS, N_PAD), 0)
    oh = (g == bat[None, :]).astype(jnp.float32)
    sums = jnp.dot(oh, z, preferred_element_type=jnp.float32)
    cnt = jnp.sum(oh, axis=1, keepdims=True)
    pooled = sums / jnp.maximum(cnt, 1.0)
    o_ref[...] = jax.nn.sigmoid(
        jnp.dot(pooled, wfc_ref[...], preferred_element_type=jnp.float32)
        + bfc_ref[...])


def _row_blk(*trail):
    return pl.BlockSpec((RB,) + trail, lambda i: (i,) + (0,) * len(trail))


def _pair_blk(d):
    return pl.BlockSpec((2, RB, d), lambda i: (0, i, 0))


def _full(shape):
    return pl.BlockSpec(shape, lambda i: (0,) * len(shape))


def kernel(x, edge_index, batch, W1, b1, W2, b2, W3, b3, Wfc, bfc):
    ei = edge_index.astype(jnp.int32)
    bat = batch.astype(jnp.int32)

    degp = _deg_kernel()(ei, _Z16)

    p1, dinv16 = pl.pallas_call(
        _tc_pre,
        grid=(GB,),
        in_specs=[_pair_blk(16), _row_blk(128), _full((128, 64))],
        out_specs=[_row_blk(64), _row_blk(16)],
        out_shape=[jax.ShapeDtypeStruct((N_PAD, 64), jnp.bfloat16),
                   jax.ShapeDtypeStruct((N_PAD, 16), jnp.float32)],
    )(degp, x, W1)
    a1 = _agg_kernel(64)(ei, p1, _Z64B)
    p2 = pl.pallas_call(
        _tc_mid,
        grid=(GB,),
        in_specs=[_row_blk(16), _pair_blk(64), _row_blk(64),
                  _full((64,)), _full((64, 32))],
        out_specs=_row_blk(32),
        out_shape=jax.ShapeDtypeStruct((N_PAD, 32), jnp.bfloat16),
    )(dinv16, a1, p1, b1, W2)
    a2 = _agg_kernel(32)(ei, p2, _Z32B)
    p3 = pl.pallas_call(
        _tc_mid,
        grid=(GB,),
        in_specs=[_row_blk(16), _pair_blk(32), _row_blk(32),
                  _full((32,)), _full((32, 16))],
        out_specs=_row_blk(16),
        out_shape=jax.ShapeDtypeStruct((N_PAD, 16), jnp.bfloat16),
    )(dinv16, a2, p2, b2, W3)
    a3 = _agg_kernel(16)(ei, p3, _Z16B)
    out = pl.pallas_call(
        _tc_fin, out_shape=jax.ShapeDtypeStruct((NUM_GRAPHS, 1), jnp.float32),
    )(dinv16, a3, p3, b3, bat, Wfc, bfc)
    return out
